# Initial kernel scaffold; baseline (speedup 1.0000x reference)
import jax
import jax.numpy as jnp
from jax.experimental import pallas as pl


def kernel(atom_features, distance, distance_membership_i, distance_membership_j, W_cf, W_df, W_fc, b_cf, b_df):
    raise NotImplementedError("write your pallas kernel here")


# speedup vs baseline: 2.1095x; 2.1095x over previous
import jax
import jax.numpy as jnp
from jax import lax
from jax.experimental import pallas as pl
from jax.experimental.pallas import tpu as pltpu
from jax.experimental.pallas import tpu_sc as plsc

N_NODES = 50000
N_EDGES = 800000
N_EMB = 30
N_DIST = 100
N_HID = 60
DP = 32
DH = 16

NC, NS = 2, 16
NW = NC * NS

G_ROWS = 6400
G_ROWS_W = G_ROWS // NW
G_CHUNK = 8
G_NCHUNK = G_ROWS_W // G_CHUNK

S_MAIN_EDGES = 6248 * 128
S_GROUPS = 6248 // 8
S_GBASE = S_GROUPS // NS
S_GEXTRA = S_GROUPS - S_GBASE * NS
NODES_PAD = 50048
NODES_T = NODES_PAD // NS


def _sc_mesh():
    return plsc.VectorSubcoreMesh(core_axis_name="c", subcore_axis_name="s",
                                  num_cores=NC, num_subcores=NS)


def _gather_body(table, idx2, out, idx_v, rows_v, sem):
    c = lax.axis_index("c")
    s = lax.axis_index("s")
    w = s * NC + c
    row0 = w * G_ROWS_W

    def chunk(t, carry):
        r0 = row0 + t * G_CHUNK
        pltpu.sync_copy(idx2.at[pl.ds(r0, G_CHUNK)], idx_v)
        descs = [
            pltpu.async_copy(table.at[idx_v.at[j]],
                             rows_v.at[pl.ds(j * 128, 128)], sem)
            for j in range(G_CHUNK)
        ]
        for d in descs:
            d.wait()
        pltpu.sync_copy(rows_v, out.at[pl.ds(r0 * 128, G_CHUNK * 128)])
        return carry

    lax.fori_loop(0, G_NCHUNK, chunk, 0)


def _scatter_body(o2, mi_main, mi_tail, z_hbm, out, idx_v, rows_v, tail_v, acc):
    c = lax.axis_index("c")
    s = lax.axis_index("s")
    pltpu.sync_copy(z_hbm.at[pl.ds(s * NODES_T, NODES_T)],
                    acc.at[pl.ds(s * NODES_T, NODES_T)])
    plsc.subcore_barrier()

    n_s = jnp.where(s < S_GEXTRA, S_GBASE + 1, S_GBASE)

    def group(t, carry):
        g = s + t * NS
        pltpu.sync_copy(mi_main.at[pl.ds(8 * g, 8)], idx_v)
        pltpu.sync_copy(o2.at[c, pl.ds(1024 * g, 1024)], rows_v)
        for j in range(8):
            pltpu.sync_copy(rows_v.at[pl.ds(j * 128, 128)],
                            acc.at[idx_v.at[j]], add=True)
        return carry

    lax.fori_loop(0, n_s, group, 0)

    @pl.when(s == 0)
    def _():
        pltpu.sync_copy(mi_tail, tail_v)
        pltpu.sync_copy(o2.at[c, pl.ds(S_MAIN_EDGES, 256)],
                        rows_v.at[pl.ds(0, 256)])
        for j in range(8):
            pltpu.sync_copy(rows_v.at[pl.ds(j * 32, 32)],
                            acc.at[tail_v.at[j]], add=True)

    plsc.subcore_barrier()
    pltpu.sync_copy(acc.at[pl.ds(s * NODES_T, NODES_T)],
                    out.at[c, pl.ds(s * NODES_T, NODES_T)])


def _edge_body(d_ref, g_ref, wdf, wcf, wfc_lo, wfc_hi, bdf, bcf, o_ref):
    dh = jnp.dot(d_ref[...], wdf[...], preferred_element_type=jnp.float32)
    dh = dh + bdf[...]
    gh = jnp.dot(g_ref[...], wcf[...], preferred_element_type=jnp.float32)
    gh = gh + bcf[...]
    p = dh * gh
    o_ref[0] = jnp.tanh(jnp.dot(p, wfc_lo[...],
                                preferred_element_type=jnp.float32))
    o_ref[1] = jnp.tanh(jnp.dot(p, wfc_hi[...],
                                preferred_element_type=jnp.float32))


def _final_body(p_ref, af_ref, wcf, wfc, bcf, bdf, out_ref):
    af = af_ref[...]
    afh = jnp.dot(af, wcf[...], preferred_element_type=jnp.float32) + bcf[...]
    self_t = jnp.tanh(jnp.dot(afh * bdf[...], wfc[...],
                              preferred_element_type=jnp.float32))
    agg = jnp.concatenate([p_ref[0], p_ref[1]], axis=-1)
    out_ref[...] = agg + af - self_t


def kernel(atom_features, distance, distance_membership_i,
           distance_membership_j, W_cf, W_df, W_fc, b_cf, b_df):
    f32 = jnp.float32
    mi = distance_membership_i.astype(jnp.int32)
    mi_main = mi[:S_MAIN_EDGES].reshape(S_MAIN_EDGES // 128, 128)
    mi_tail = mi[S_MAIN_EDGES:].reshape(8, 32)
    mj = distance_membership_j.astype(jnp.int32)
    mj2 = jnp.pad(mj, (0, G_ROWS * 128 - N_EDGES)).reshape(G_ROWS, 128)
    af_pad = jnp.pad(atom_features, ((0, 0), (0, DP - N_EMB)))
    wcf_p = jnp.pad(W_cf, ((0, DP - N_EMB), (0, 0)))
    wfc_p = jnp.pad(W_fc, ((0, 0), (0, DP - N_EMB)))
    wfc_lo = wfc_p[:, :DH]
    wfc_hi = wfc_p[:, DH:]
    bcf2 = b_cf.reshape(1, N_HID)
    bdf2 = b_df.reshape(1, N_HID)

    gather_call = pl.kernel(
        _gather_body,
        out_type=jax.ShapeDtypeStruct((G_ROWS * 128, DP), f32),
        mesh=_sc_mesh(),
        scratch_types=[
            pltpu.VMEM((G_CHUNK, 128), jnp.int32),
            pltpu.VMEM((G_CHUNK * 128, DP), f32),
            pltpu.SemaphoreType.DMA,
        ],
        compiler_params=pltpu.CompilerParams(use_tc_tiling_on_sc=False),
    )
    g0 = gather_call(af_pad, mj2)

    EB = 5000
    o2 = pl.pallas_call(
        _edge_body,
        grid=(N_EDGES // EB,),
        in_specs=[
            pl.BlockSpec((EB, N_DIST), lambda i: (i, 0)),
            pl.BlockSpec((EB, DP), lambda i: (i, 0)),
            pl.BlockSpec((N_DIST, N_HID), lambda i: (0, 0)),
            pl.BlockSpec((DP, N_HID), lambda i: (0, 0)),
            pl.BlockSpec((N_HID, DH), lambda i: (0, 0)),
            pl.BlockSpec((N_HID, DH), lambda i: (0, 0)),
            pl.BlockSpec((1, N_HID), lambda i: (0, 0)),
            pl.BlockSpec((1, N_HID), lambda i: (0, 0)),
        ],
        out_specs=pl.BlockSpec((NC, EB, DH), lambda i: (0, i, 0)),
        out_shape=jax.ShapeDtypeStruct((NC, N_EDGES, DH), f32),
    )(distance, g0, W_df, wcf_p, wfc_lo, wfc_hi, bdf2, bcf2)

    z = jnp.zeros((NODES_PAD, DH), f32)
    scatter_call = pl.kernel(
        _scatter_body,
        out_type=jax.ShapeDtypeStruct((NC, NODES_PAD, DH), f32),
        mesh=_sc_mesh(),
        scratch_types=[
            pltpu.VMEM((8, 128), jnp.int32),
            pltpu.VMEM((1024, DH), f32),
            pltpu.VMEM((8, 32), jnp.int32),
            pltpu.VMEM_SHARED((NODES_PAD, DH), f32),
        ],
        compiler_params=pltpu.CompilerParams(use_tc_tiling_on_sc=False),
    )
    partials = scatter_call(o2, mi_main, mi_tail, z)

    NB = 5000
    out_pad = pl.pallas_call(
        _final_body,
        grid=(N_NODES // NB,),
        in_specs=[
            pl.BlockSpec((NC, NB, DH), lambda i: (0, i, 0)),
            pl.BlockSpec((NB, DP), lambda i: (i, 0)),
            pl.BlockSpec((DP, N_HID), lambda i: (0, 0)),
            pl.BlockSpec((N_HID, DP), lambda i: (0, 0)),
            pl.BlockSpec((1, N_HID), lambda i: (0, 0)),
            pl.BlockSpec((1, N_HID), lambda i: (0, 0)),
        ],
        out_specs=pl.BlockSpec((NB, DP), lambda i: (i, 0)),
        out_shape=jax.ShapeDtypeStruct((N_NODES, DP), f32),
    )(partials, af_pad, wcf_p, wfc_p, bcf2, bdf2)

    return out_pad[:, :N_EMB]


# speedup vs baseline: 4.1104x; 1.9486x over previous
import jax
import jax.numpy as jnp
from jax import lax
from jax.experimental import pallas as pl
from jax.experimental.pallas import tpu as pltpu
from jax.experimental.pallas import tpu_sc as plsc

N_NODES = 50000
N_EDGES = 800000
N_EMB = 30
N_DIST = 100
N_HID = 60
DP = 32

NC, NS = 2, 16
NW = NC * NS

G_ROWS = 6400
G_ROWS_W = G_ROWS // NW
G_CHUNK = 8
G_NCHUNK = G_ROWS_W // G_CHUNK

EB = 6400
GRID = N_EDGES // EB
WIN = 256
ACC_L = 50176


def _sc_mesh():
    return plsc.VectorSubcoreMesh(core_axis_name="c", subcore_axis_name="s",
                                  num_cores=NC, num_subcores=NS)


def _gather_body(table, idx2, out, idx_v, rows_v, sem):
    c = lax.axis_index("c")
    s = lax.axis_index("s")
    w = s * NC + c
    row0 = w * G_ROWS_W

    def chunk(t, carry):
        r0 = row0 + t * G_CHUNK
        pltpu.sync_copy(idx2.at[pl.ds(r0, G_CHUNK)], idx_v)
        descs = [
            pltpu.async_copy(table.at[idx_v.at[j]],
                             rows_v.at[pl.ds(j * 128, 128)], sem)
            for j in range(G_CHUNK)
        ]
        for d in descs:
            d.wait()
        pltpu.sync_copy(rows_v, out.at[pl.ds(r0 * 128, G_CHUNK * 128)])
        return carry

    lax.fori_loop(0, G_NCHUNK, chunk, 0)


def _mega_body(dt_ref, g_ref, mi_ref, aft_ref, wdf_t, wcf_tp, wcf_t, wfc_t,
               bdf_t, bcf_t, out_ref, acc_ref):
    i = pl.program_id(0)
    f32 = jnp.float32

    @pl.when(i == 0)
    def _():
        acc_ref[...] = jnp.zeros((N_EMB, ACC_L), f32)

    dh_t = jnp.dot(wdf_t[...], dt_ref[...], preferred_element_type=f32)
    dh_t = dh_t + bdf_t[...]
    g_t = jnp.swapaxes(g_ref[...], 0, 1)
    gh_t = jnp.dot(wcf_tp[...], g_t, preferred_element_type=f32) + bcf_t[...]
    p_t = dh_t * gh_t
    o_t = jnp.tanh(jnp.dot(wfc_t[...], p_t, preferred_element_type=f32))

    ids = mi_ref[...].reshape(1, EB)
    lo = jnp.min(ids)
    hi = jnp.max(ids)
    base0 = (lo // 128) * 128
    nwin = (hi - base0) // WIN + 1

    def win(t, carry):
        wb = base0 + t * WIN
        iota = lax.broadcasted_iota(jnp.int32, (WIN, EB), 0) + wb
        oh = (iota == ids).astype(f32)
        partial = lax.dot_general(o_t, oh, (((1,), (1,)), ((), ())),
                                  preferred_element_type=f32)
        acc_ref[:, pl.ds(wb, WIN)] += partial
        return carry

    lax.fori_loop(0, nwin, win, 0)

    @pl.when(i == GRID - 1)
    def _():
        aft = aft_ref[...]
        afh_t = jnp.dot(wcf_t[...], aft, preferred_element_type=f32)
        afh_t = afh_t + bcf_t[...]
        self_t = jnp.tanh(jnp.dot(wfc_t[...], afh_t * bdf_t[...],
                                  preferred_element_type=f32))
        out_ref[...] = acc_ref[:, :N_NODES] + aft - self_t


def kernel(atom_features, distance, distance_membership_i,
           distance_membership_j, W_cf, W_df, W_fc, b_cf, b_df):
    f32 = jnp.float32
    mi3 = distance_membership_i.astype(jnp.int32).reshape(GRID, 1, EB)
    mj = distance_membership_j.astype(jnp.int32)
    mj2 = jnp.pad(mj, (0, G_ROWS * 128 - N_EDGES)).reshape(G_ROWS, 128)
    af_pad = jnp.pad(atom_features, ((0, 0), (0, DP - N_EMB)))
    dist_t = distance.T
    af_t = atom_features.T
    wdf_t = W_df.T
    wcf_tp = jnp.pad(W_cf, ((0, DP - N_EMB), (0, 0))).T
    wcf_t = W_cf.T
    wfc_t = W_fc.T
    bdf_t = b_df.reshape(N_HID, 1)
    bcf_t = b_cf.reshape(N_HID, 1)

    gather_call = pl.kernel(
        _gather_body,
        out_type=jax.ShapeDtypeStruct((G_ROWS * 128, DP), f32),
        mesh=_sc_mesh(),
        scratch_types=[
            pltpu.VMEM((G_CHUNK, 128), jnp.int32),
            pltpu.VMEM((G_CHUNK * 128, DP), f32),
            pltpu.SemaphoreType.DMA,
        ],
        compiler_params=pltpu.CompilerParams(use_tc_tiling_on_sc=False),
    )
    g0 = gather_call(af_pad, mj2)

    out_t = pl.pallas_call(
        _mega_body,
        grid=(GRID,),
        in_specs=[
            pl.BlockSpec((N_DIST, EB), lambda i: (0, i)),
            pl.BlockSpec((EB, DP), lambda i: (i, 0)),
            pl.BlockSpec((1, 1, EB), lambda i: (i, 0, 0)),
            pl.BlockSpec((N_EMB, N_NODES), lambda i: (0, 0)),
            pl.BlockSpec((N_HID, N_DIST), lambda i: (0, 0)),
            pl.BlockSpec((N_HID, DP), lambda i: (0, 0)),
            pl.BlockSpec((N_HID, N_EMB), lambda i: (0, 0)),
            pl.BlockSpec((N_EMB, N_HID), lambda i: (0, 0)),
            pl.BlockSpec((N_HID, 1), lambda i: (0, 0)),
            pl.BlockSpec((N_HID, 1), lambda i: (0, 0)),
        ],
        out_specs=pl.BlockSpec((N_EMB, N_NODES), lambda i: (0, 0)),
        out_shape=jax.ShapeDtypeStruct((N_EMB, N_NODES), f32),
        scratch_shapes=[pltpu.VMEM((N_EMB, ACC_L), f32)],
    )(dist_t, g0, mi3, af_t, wdf_t, wcf_tp, wcf_t, wfc_t, bdf_t, bcf_t)

    return out_t.T


# speedup vs baseline: 4.6423x; 1.1294x over previous
import jax
import jax.numpy as jnp
from jax import lax
from jax.experimental import pallas as pl
from jax.experimental.pallas import tpu as pltpu
from jax.experimental.pallas import tpu_sc as plsc

N_NODES = 50000
N_EDGES = 800000
N_EMB = 30
N_DIST = 100
N_HID = 60
DP = 32

NC, NS = 2, 16
NW = NC * NS

G_ROWS = 6400
G_ROWS_W = G_ROWS // NW
G_CHUNK = 25
G_NCHUNK = G_ROWS_W // G_CHUNK

EB = 6400
GRID = N_EDGES // EB
WIN = 256
ACC_L = 50176


def _sc_mesh():
    return plsc.VectorSubcoreMesh(core_axis_name="c", subcore_axis_name="s",
                                  num_cores=NC, num_subcores=NS)


def _gather_body(table, idx2, out, idx_v, rows_v, sem):
    c = lax.axis_index("c")
    s = lax.axis_index("s")
    w = s * NC + c
    row0 = w * G_ROWS_W

    def chunk(t, carry):
        r0 = row0 + t * G_CHUNK
        pltpu.sync_copy(idx2.at[pl.ds(r0, G_CHUNK)], idx_v)
        descs = [
            pltpu.async_copy(table.at[idx_v.at[j]],
                             rows_v.at[pl.ds(j * 128, 128)], sem)
            for j in range(G_CHUNK)
        ]
        for d in descs:
            d.wait()
        pltpu.sync_copy(rows_v, out.at[pl.ds(r0 * 128, G_CHUNK * 128)])
        return carry

    lax.fori_loop(0, G_NCHUNK, chunk, 0)


def _mega_body(dt_ref, g_ref, mi_ref, aft_ref, wdf_t, wcf_tp, wcf_t, wfc_t,
               bdf_t, bcf_t, out_ref, acc_ref):
    i = pl.program_id(0)
    f32 = jnp.float32

    @pl.when(i == 0)
    def _():
        acc_ref[...] = jnp.zeros((N_EMB, ACC_L), f32)

    dh_t = jnp.dot(wdf_t[...], dt_ref[...], preferred_element_type=f32)
    dh_t = dh_t + bdf_t[...]
    g_t = jnp.swapaxes(g_ref[...], 0, 1).astype(f32)
    gh_t = jnp.dot(wcf_tp[...], g_t, preferred_element_type=f32) + bcf_t[...]
    p_t = dh_t * gh_t
    o_t = jnp.tanh(jnp.dot(wfc_t[...], p_t, preferred_element_type=f32))

    ids = mi_ref[...].reshape(1, EB)
    lo = jnp.min(ids)
    hi = jnp.max(ids)
    base0 = (lo // 128) * 128
    nwin = (hi - base0) // WIN + 1

    def win(t, carry):
        wb = base0 + t * WIN
        iota = lax.broadcasted_iota(jnp.int32, (WIN, EB), 0) + wb
        oh = (iota == ids).astype(f32)
        partial = lax.dot_general(o_t, oh, (((1,), (1,)), ((), ())),
                                  preferred_element_type=f32)
        acc_ref[:, pl.ds(wb, WIN)] += partial
        return carry

    lax.fori_loop(0, nwin, win, 0)

    @pl.when(i == GRID - 1)
    def _():
        aft = aft_ref[...]
        afh_t = jnp.dot(wcf_t[...], aft, preferred_element_type=f32)
        afh_t = afh_t + bcf_t[...]
        self_t = jnp.tanh(jnp.dot(wfc_t[...], afh_t * bdf_t[...],
                                  preferred_element_type=f32))
        out_ref[...] = acc_ref[:, :N_NODES] + aft - self_t


def kernel(atom_features, distance, distance_membership_i,
           distance_membership_j, W_cf, W_df, W_fc, b_cf, b_df):
    f32 = jnp.float32
    mi3 = distance_membership_i.astype(jnp.int32).reshape(GRID, 1, EB)
    mj = distance_membership_j.astype(jnp.int32)
    mj2 = jnp.pad(mj, (0, G_ROWS * 128 - N_EDGES)).reshape(G_ROWS, 128)
    af_pad = jnp.pad(atom_features, ((0, 0), (0, DP - N_EMB)))
    dist_t = distance.T
    af_t = atom_features.T
    wdf_t = W_df.T
    wcf_tp = jnp.pad(W_cf, ((0, DP - N_EMB), (0, 0))).T
    wcf_t = W_cf.T
    wfc_t = W_fc.T
    bdf_t = b_df.reshape(N_HID, 1)
    bcf_t = b_cf.reshape(N_HID, 1)

    gather_call = pl.kernel(
        _gather_body,
        out_type=jax.ShapeDtypeStruct((G_ROWS * 128, DP), jnp.bfloat16),
        mesh=_sc_mesh(),
        scratch_types=[
            pltpu.VMEM((G_CHUNK, 128), jnp.int32),
            pltpu.VMEM((G_CHUNK * 128, DP), jnp.bfloat16),
            pltpu.SemaphoreType.DMA,
        ],
        compiler_params=pltpu.CompilerParams(use_tc_tiling_on_sc=False),
    )
    g0 = gather_call(af_pad.astype(jnp.bfloat16), mj2)

    out_t = pl.pallas_call(
        _mega_body,
        grid=(GRID,),
        in_specs=[
            pl.BlockSpec((N_DIST, EB), lambda i: (0, i)),
            pl.BlockSpec((EB, DP), lambda i: (i, 0)),
            pl.BlockSpec((1, 1, EB), lambda i: (i, 0, 0)),
            pl.BlockSpec((N_EMB, N_NODES), lambda i: (0, 0)),
            pl.BlockSpec((N_HID, N_DIST), lambda i: (0, 0)),
            pl.BlockSpec((N_HID, DP), lambda i: (0, 0)),
            pl.BlockSpec((N_HID, N_EMB), lambda i: (0, 0)),
            pl.BlockSpec((N_EMB, N_HID), lambda i: (0, 0)),
            pl.BlockSpec((N_HID, 1), lambda i: (0, 0)),
            pl.BlockSpec((N_HID, 1), lambda i: (0, 0)),
        ],
        out_specs=pl.BlockSpec((N_EMB, N_NODES), lambda i: (0, 0)),
        out_shape=jax.ShapeDtypeStruct((N_EMB, N_NODES), f32),
        scratch_shapes=[pltpu.VMEM((N_EMB, ACC_L), f32)],
    )(dist_t, g0, mi3, af_t, wdf_t, wcf_tp, wcf_t, wfc_t, bdf_t, bcf_t)

    return out_t.T
